# baseline (speedup 1.0000x reference)
import functools

import jax
import jax.numpy as jnp
from jax import lax
from jax.experimental import pallas as pl
from jax.experimental.pallas import tpu as pltpu
from jax.experimental.pallas import tpu_sc as plsc

NC, NS = 2, 16
NW = NC * NS
D = 128
PP = 20
NN = 200
RM = 240
SL = RM // 2
NBUF = 2


def _sc_dots(combined_idx, input_labels, in_embed, out_embed, B):
    b_per_w = B // NW
    mesh = plsc.VectorSubcoreMesh(core_axis_name="c", subcore_axis_name="s")

    @functools.partial(
        pl.kernel,
        mesh=mesh,
        compiler_params=pltpu.CompilerParams(needs_layout_passes=False),
        out_type=jax.ShapeDtypeStruct((B, RM), jnp.float32),
        scratch_types=[
            pltpu.VMEM((b_per_w,), jnp.int32),
            pltpu.VMEM((b_per_w, D), jnp.float32),
            pltpu.VMEM((b_per_w * RM,), jnp.int32),
            pltpu.VMEM((NBUF * RM, D), jnp.float32),
            pltpu.VMEM((RM,), jnp.float32),
            pltpu.VMEM((RM,), jnp.float32),
            pltpu.SemaphoreType.DMA,
            pltpu.SemaphoreType.DMA,
            pltpu.SemaphoreType.DMA,
            pltpu.SemaphoreType.DMA,
        ],
    )
    def k(idx_hbm, cidx_hbm, in_tab, out_tab, out_hbm,
          cidx_v, cent_v, idx_v, rows_v, dots0_v, dots1_v, s0, s1, d0, d1):
        wid = lax.axis_index("s") * NC + lax.axis_index("c")
        base = wid * b_per_w
        pltpu.sync_copy(cidx_hbm.at[pl.ds(base, b_per_w)], cidx_v)
        pltpu.sync_copy(idx_hbm.at[pl.ds(base * RM, b_per_w * RM)], idx_v)
        pltpu.async_copy(in_tab.at[cidx_v], cent_v, s0).wait()
        lane = lax.iota(jnp.int32, 16)
        m15 = lane == 15

        def fire(b, buf, sem):
            off = pl.multiple_of(b * RM, 8)
            dst = buf * RM
            pltpu.async_copy(
                out_tab.at[idx_v.at[pl.ds(off, SL)]],
                rows_v.at[pl.ds(dst, SL)], sem)
            pltpu.async_copy(
                out_tab.at[idx_v.at[pl.ds(off + SL, SL)]],
                rows_v.at[pl.ds(dst + SL, SL)], sem)

        def wait_rows(sem, buf):
            pltpu.make_async_copy(
                out_tab.at[pl.ds(0, RM)],
                rows_v.at[pl.ds(buf * RM, RM)], sem).wait()

        def wait_dots(dots_ref, sem):
            pltpu.make_async_copy(dots_ref, out_hbm.at[base], sem).wait()

        def compute(buf, bl, dots_ref):
            rbase = buf * RM
            cs = [cent_v[bl, pl.ds(16 * j, 16)] for j in range(8)]

            def per16(g, c2):
                gbase = jnp.full((16,), g * 16, jnp.int32)
                for kk in range(16):
                    row = rbase + g * 16 + kk
                    p = [rows_v[row, pl.ds(16 * j, 16)] * cs[j]
                         for j in range(8)]
                    acc = (((p[0] + p[1]) + (p[2] + p[3]))
                           + ((p[4] + p[5]) + (p[6] + p[7])))
                    s = plsc.cumsum(acc)
                    plsc.store_scatter(dots_ref, [gbase + kk], s, mask=m15)
                return c2

            lax.fori_loop(0, RM // 16, per16, 0)

        bufs = ((0, s0, d0, dots0_v), (1, s1, d1, dots1_v))
        for buf, rs, _, _ in bufs:
            fire(buf, buf, rs)
        nt = b_per_w // NBUF

        def body(t, carry):
            for kk, (buf, rs, dsm, dref) in enumerate(bufs):
                b = NBUF * t + kk
                wait_rows(rs, buf)

                @pl.when(t > 0)
                def _():
                    wait_dots(dref, dsm)

                compute(buf, b, dref)

                @pl.when(b + NBUF < b_per_w)
                def _():
                    fire(b + NBUF, buf, rs)

                pltpu.async_copy(dref, out_hbm.at[base + b], dsm)
            return carry

        lax.fori_loop(0, nt, body, 0)
        for kk in range(b_per_w - NBUF * nt):
            buf, rs, dsm, dref = bufs[kk]
            b = NBUF * nt + kk
            wait_rows(rs, buf)
            wait_dots(dref, dsm)
            compute(buf, b, dref)
            pltpu.async_copy(dref, out_hbm.at[base + b], dsm)
        for _, _, dsm, dref in bufs:
            wait_dots(dref, dsm)

    return k(combined_idx, input_labels, in_embed, out_embed)


def _tc_loss(dots, B):
    bblk = 512

    def body(d_ref, o_ref):
        x = d_ref[...]
        col = lax.broadcasted_iota(jnp.int32, x.shape, 1)
        y = jnp.where(col < PP, x, -x)
        ls = jax.nn.log_sigmoid(y)
        ls = jnp.where(col < PP + NN, ls, 0.0)
        o_ref[...] = -jnp.sum(ls, axis=1)

    return pl.pallas_call(
        body,
        grid=(B // bblk,),
        in_specs=[pl.BlockSpec((bblk, RM), lambda i: (i, 0))],
        out_specs=pl.BlockSpec((bblk,), lambda i: (i,)),
        out_shape=jax.ShapeDtypeStruct((B,), jnp.float32),
    )(dots)


def kernel(input_labels, pos_labels, neg_labels, in_embed, out_embed):
    B = input_labels.shape[0]
    pad = jnp.zeros((B, RM - PP - NN), jnp.int32)
    combined = jnp.concatenate(
        [pos_labels, neg_labels, pad], axis=1).reshape(-1)
    dots = _sc_dots(combined, input_labels, in_embed, out_embed, B)
    return _tc_loss(dots, B)


# speedup vs baseline: 3.8453x; 3.8453x over previous
import functools

import jax
import jax.numpy as jnp
from jax import lax
from jax.experimental import pallas as pl
from jax.experimental.pallas import tpu as pltpu
from jax.experimental.pallas import tpu_sc as plsc

NC, NS = 2, 16
NW = NC * NS
D = 128
PP = 20
NN = 200
RM = 224
SL = RM // 2
NBUF = 2


def _sc_dots(combined_idx, input_labels, in_embed, out_embed, B):
    b_per_w = B // NW
    mesh = plsc.VectorSubcoreMesh(core_axis_name="c", subcore_axis_name="s")

    @functools.partial(
        pl.kernel,
        mesh=mesh,
        compiler_params=pltpu.CompilerParams(needs_layout_passes=False),
        out_type=jax.ShapeDtypeStruct((B, RM), jnp.float32),
        scratch_types=[
            pltpu.VMEM((b_per_w,), jnp.int32),
            pltpu.VMEM((b_per_w, D), jnp.float32),
            pltpu.VMEM((b_per_w * RM,), jnp.int32),
            pltpu.VMEM((NBUF * RM, D), jnp.float32),
            pltpu.VMEM((RM,), jnp.float32),
            pltpu.VMEM((RM,), jnp.float32),
            pltpu.SemaphoreType.DMA,
            pltpu.SemaphoreType.DMA,
            pltpu.SemaphoreType.DMA,
            pltpu.SemaphoreType.DMA,
        ],
    )
    def k(idx_hbm, cidx_hbm, in_tab, out_tab, out_hbm,
          cidx_v, cent_v, idx_v, rows_v, dots0_v, dots1_v, s0, s1, d0, d1):
        wid = lax.axis_index("s") * NC + lax.axis_index("c")
        base = wid * b_per_w
        pltpu.sync_copy(cidx_hbm.at[pl.ds(base, b_per_w)], cidx_v)
        pltpu.sync_copy(idx_hbm.at[pl.ds(base * RM, b_per_w * RM)], idx_v)
        pltpu.async_copy(in_tab.at[cidx_v], cent_v, s0).wait()
        lane = lax.iota(jnp.int32, 16)
        m15 = lane == 15

        def fire(b, buf, sem):
            off = pl.multiple_of(b * RM, 8)
            dst = buf * RM
            pltpu.async_copy(
                out_tab.at[idx_v.at[pl.ds(off, SL)]],
                rows_v.at[pl.ds(dst, SL)], sem)
            pltpu.async_copy(
                out_tab.at[idx_v.at[pl.ds(off + SL, SL)]],
                rows_v.at[pl.ds(dst + SL, SL)], sem)

        def wait_rows(sem, buf):
            pltpu.make_async_copy(
                out_tab.at[pl.ds(0, RM)],
                rows_v.at[pl.ds(buf * RM, RM)], sem).wait()

        def wait_dots(dots_ref, sem):
            pltpu.make_async_copy(dots_ref, out_hbm.at[base], sem).wait()

        def compute(buf, bl, dots_ref):
            rbase = buf * RM
            cs = [cent_v[bl, pl.ds(16 * j, 16)] for j in range(8)]

            def per16(g, c2):
                gbase = jnp.full((16,), g * 16, jnp.int32)
                for kk in range(16):
                    row = rbase + g * 16 + kk
                    p = [rows_v[row, pl.ds(16 * j, 16)] * cs[j]
                         for j in range(8)]
                    acc = (((p[0] + p[1]) + (p[2] + p[3]))
                           + ((p[4] + p[5]) + (p[6] + p[7])))
                    s = plsc.cumsum(acc)
                    plsc.store_scatter(dots_ref, [gbase + kk], s, mask=m15)
                return c2

            lax.fori_loop(0, RM // 16, per16, 0)

        bufs = ((0, s0, d0, dots0_v), (1, s1, d1, dots1_v))
        for buf, rs, _, _ in bufs:
            fire(buf, buf, rs)
        nt = b_per_w // NBUF

        def body(t, carry):
            for kk, (buf, rs, dsm, dref) in enumerate(bufs):
                b = NBUF * t + kk
                wait_rows(rs, buf)

                @pl.when(t > 0)
                def _():
                    wait_dots(dref, dsm)

                compute(buf, b, dref)

                @pl.when(b + NBUF < b_per_w)
                def _():
                    fire(b + NBUF, buf, rs)

                pltpu.async_copy(dref, out_hbm.at[base + b], dsm)
            return carry

        lax.fori_loop(0, nt, body, 0)
        for kk in range(b_per_w - NBUF * nt):
            buf, rs, dsm, dref = bufs[kk]
            b = NBUF * nt + kk
            wait_rows(rs, buf)
            wait_dots(dref, dsm)
            compute(buf, b, dref)
            pltpu.async_copy(dref, out_hbm.at[base + b], dsm)
        for _, _, dsm, dref in bufs:
            wait_dots(dref, dsm)

    return k(combined_idx, input_labels, in_embed, out_embed)


def _tc_loss(dots, B):
    bblk = 512

    def body(d_ref, o_ref):
        x = d_ref[...]
        col = lax.broadcasted_iota(jnp.int32, x.shape, 1)
        y = jnp.where(col < PP, x, -x)
        ls = jax.nn.log_sigmoid(y)
        ls = jnp.where(col < PP + NN, ls, 0.0)
        o_ref[...] = -jnp.sum(ls, axis=1)

    return pl.pallas_call(
        body,
        grid=(B // bblk,),
        in_specs=[pl.BlockSpec((bblk, RM), lambda i: (i, 0))],
        out_specs=pl.BlockSpec((bblk,), lambda i: (i,)),
        out_shape=jax.ShapeDtypeStruct((B,), jnp.float32),
    )(dots)


def kernel(input_labels, pos_labels, neg_labels, in_embed, out_embed):
    B = input_labels.shape[0]
    pad = jnp.zeros((B, RM - PP - NN), jnp.int32)
    combined = jnp.concatenate(
        [pos_labels, neg_labels, pad], axis=1).reshape(-1)
    dots = _sc_dots(combined, input_labels, in_embed, out_embed, B)
    return _tc_loss(dots, B)
